# Initial kernel scaffold; baseline (speedup 1.0000x reference)
import jax
import jax.numpy as jnp
from jax.experimental import pallas as pl


def kernel(input):
    raise NotImplementedError("write your pallas kernel here")


# speedup vs baseline: 18.5852x; 18.5852x over previous
import functools

import jax
import jax.numpy as jnp
from jax import lax
from jax.experimental import pallas as pl
from jax.experimental.pallas import tpu as pltpu
from jax.experimental.pallas import tpu_sc as plsc

_B = 128
_N = 32768
_L = 16
_NSLICES = _N // _L
_NWORKERS = 32
_ROWS_PER_WORKER = _B // _NWORKERS

_mesh = plsc.VectorSubcoreMesh(core_axis_name="c", subcore_axis_name="s")


@functools.partial(
    pl.kernel,
    out_type=jax.ShapeDtypeStruct((_B, _N), jnp.float32),
    mesh=_mesh,
    scratch_types=[
        pltpu.VMEM((_N,), jnp.float32),
        pltpu.VMEM((_N + _L,), jnp.float32),
    ],
    compiler_params=pltpu.CompilerParams(needs_layout_passes=False),
)
def _sparsemax_sc(x_hbm, out_hbm, row_v, cand_v):
    wid = lax.axis_index("s") * 2 + lax.axis_index("c")
    iota = lax.broadcasted_iota(jnp.int32, (_L,), 0)

    for r in range(_ROWS_PER_WORKER):
        row = wid * _ROWS_PER_WORKER + r
        pltpu.sync_copy(x_hbm.at[row], row_v)

        def max_body(i, vmax):
            return jnp.maximum(vmax, row_v[pl.ds(i * _L, _L)])

        vmax = lax.fori_loop(
            0, _NSLICES, max_body, jnp.full((_L,), -3.0e38, jnp.float32),
            unroll=8)
        m = jnp.max(vmax)
        t0 = m - 1.0

        def comp_body(i, off):
            v = row_v[pl.ds(i * _L, _L)]
            msk = v > t0
            plsc.store_compressed(cand_v.at[pl.ds(off, _L)], v, mask=msk)
            cnt = plsc.all_reduce_population_count(msk)
            return off + cnt[0]

        n = lax.fori_loop(0, _NSLICES, comp_body, jnp.int32(0), unroll=8)
        ncs = (n + _L - 1) // _L

        def fk(t):
            def body(j, acc):
                s_acc, k_acc = acc
                v = cand_v[pl.ds(j * _L, _L)]
                msk = (v > t) & ((j * _L + iota) < n)
                return (s_acc + jnp.where(msk, v, 0.0),
                        k_acc + jnp.where(msk, 1.0, 0.0))

            z = jnp.zeros((_L,), jnp.float32)
            s_acc, k_acc = lax.fori_loop(0, ncs, body, (z, z))
            return jnp.sum(s_acc), jnp.sum(k_acc)

        def mich_cond(carry):
            t_prev, t = carry
            return t > t_prev

        def mich_body(carry):
            _, t = carry
            s, k = fk(t)
            t_new = ((jnp.full((_L,), s) - 1.0) / jnp.full((_L,), k))[0]
            return (t, jnp.maximum(t, t_new))

        _, tau = lax.while_loop(mich_cond, mich_body, (t0 - 1.0, t0))

        def out_body(i, carry):
            v = row_v[pl.ds(i * _L, _L)]
            row_v[pl.ds(i * _L, _L)] = jnp.maximum(v - tau, 0.0)
            return carry

        lax.fori_loop(0, _NSLICES, out_body, 0, unroll=8)
        pltpu.sync_copy(row_v, out_hbm.at[row])


def kernel(input):
    return _sparsemax_sc(input)
